# baseline (speedup 1.0000x reference)
import jax
import jax.numpy as jnp
from jax.experimental import pallas as pl
from jax.experimental.pallas import tpu as pltpu

BM = 256
CHR = 256
N_FEAT = 4096


def _body(x_ref, w_hbm, b_ref, o_ref, w_bf, st0, st1, s0, s1):
    m = pl.program_id(0)

    @pl.when(m == 0)
    def _load_w_and_first_block():
        stages = (st0, st1)
        sems = (s0, s1)
        nch = N_FEAT // CHR
        pltpu.make_async_copy(
            w_hbm.at[pl.ds(0, CHR), :], stages[0], sems[0]).start()
        for c in range(nch):
            if c + 1 < nch:
                pltpu.make_async_copy(
                    w_hbm.at[pl.ds((c + 1) * CHR, CHR), :],
                    stages[(c + 1) % 2], sems[(c + 1) % 2]).start()
            pltpu.make_async_copy(
                w_hbm.at[pl.ds(c * CHR, CHR), :],
                stages[c % 2], sems[c % 2]).wait()
            wc = stages[c % 2][...].astype(jnp.bfloat16)
            w_bf[pl.ds(c * CHR, CHR), :] = wc
            o_ref[:, pl.ds(c * CHR, CHR)] = jax.lax.dot_general(
                x_ref[...], wc,
                (((1,), (1,)), ((), ())),
                preferred_element_type=jnp.float32) + b_ref[:, pl.ds(c * CHR, CHR)]

    @pl.when(m > 0)
    def _block():
        hk = N_FEAT // 2
        h0 = jax.lax.dot_general(
            x_ref[:, :hk], w_bf[:, :hk],
            (((1,), (1,)), ((), ())),
            preferred_element_type=jnp.float32)
        h1 = jax.lax.dot_general(
            x_ref[:, hk:], w_bf[:, hk:],
            (((1,), (1,)), ((), ())),
            preferred_element_type=jnp.float32)
        o_ref[...] = h0 + h1 + b_ref[...]


def kernel(input, W, bias):
    B, S, K = input.shape
    N = W.shape[0]
    M = B * S
    x = input.reshape(M, K)
    b2 = bias.reshape(1, N)
    nm = M // BM
    out = pl.pallas_call(
        _body,
        grid=(nm,),
        in_specs=[
            pl.BlockSpec((BM, K), lambda m: (m, 0)),
            pl.BlockSpec(memory_space=pltpu.MemorySpace.HBM),
            pl.BlockSpec((1, N), lambda m: (0, 0)),
        ],
        out_specs=pl.BlockSpec((BM, N), lambda m: (m, 0)),
        out_shape=jax.ShapeDtypeStruct((M, N), jnp.float32),
        scratch_shapes=[
            pltpu.VMEM((N, K), jnp.bfloat16),
            pltpu.VMEM((CHR, K), jnp.float32),
            pltpu.VMEM((CHR, K), jnp.float32),
            pltpu.SemaphoreType.DMA,
            pltpu.SemaphoreType.DMA,
        ],
        compiler_params=pltpu.CompilerParams(
            dimension_semantics=("arbitrary",),
            vmem_limit_bytes=100 * 1024 * 1024,
        ),
    )(x, W, b2)
    return out.reshape(B, S, N)


# speedup vs baseline: 1.0024x; 1.0024x over previous
import jax
import jax.numpy as jnp
from jax.experimental import pallas as pl
from jax.experimental.pallas import tpu as pltpu

BM = 256
CHR = 256
N_FEAT = 4096


def _body(x_ref, w_hbm, b_ref, o_ref, w_bf, st0, st1, s0, s1):
    m = pl.program_id(0)

    @pl.when(m == 0)
    def _load_w_and_first_block():
        stages = (st0, st1)
        sems = (s0, s1)
        nch = N_FEAT // CHR
        pltpu.make_async_copy(
            w_hbm.at[pl.ds(0, CHR), :], stages[0], sems[0]).start()
        for c in range(nch):
            if c + 1 < nch:
                pltpu.make_async_copy(
                    w_hbm.at[pl.ds((c + 1) * CHR, CHR), :],
                    stages[(c + 1) % 2], sems[(c + 1) % 2]).start()
            pltpu.make_async_copy(
                w_hbm.at[pl.ds(c * CHR, CHR), :],
                stages[c % 2], sems[c % 2]).wait()
            wc = stages[c % 2][...].astype(jnp.bfloat16)
            w_bf[pl.ds(c * CHR, CHR), :] = wc
            o_ref[:, pl.ds(c * CHR, CHR)] = jax.lax.dot_general(
                x_ref[...], wc,
                (((1,), (1,)), ((), ())),
                preferred_element_type=jnp.float32) + b_ref[:, pl.ds(c * CHR, CHR)]

    @pl.when(m > 0)
    def _block():
        hn = N_FEAT // 2
        o_ref[:, :hn] = jax.lax.dot_general(
            x_ref[...], w_bf[:hn, :],
            (((1,), (1,)), ((), ())),
            preferred_element_type=jnp.float32) + b_ref[:, :hn]
        o_ref[:, hn:] = jax.lax.dot_general(
            x_ref[...], w_bf[hn:, :],
            (((1,), (1,)), ((), ())),
            preferred_element_type=jnp.float32) + b_ref[:, hn:]


def kernel(input, W, bias):
    B, S, K = input.shape
    N = W.shape[0]
    M = B * S
    x = input.reshape(M, K)
    b2 = bias.reshape(1, N)
    nm = M // BM
    out = pl.pallas_call(
        _body,
        grid=(nm,),
        in_specs=[
            pl.BlockSpec((BM, K), lambda m: (m, 0)),
            pl.BlockSpec(memory_space=pltpu.MemorySpace.HBM),
            pl.BlockSpec((1, N), lambda m: (0, 0)),
        ],
        out_specs=pl.BlockSpec((BM, N), lambda m: (m, 0)),
        out_shape=jax.ShapeDtypeStruct((M, N), jnp.float32),
        scratch_shapes=[
            pltpu.VMEM((N, K), jnp.bfloat16),
            pltpu.VMEM((CHR, K), jnp.float32),
            pltpu.VMEM((CHR, K), jnp.float32),
            pltpu.SemaphoreType.DMA,
            pltpu.SemaphoreType.DMA,
        ],
        compiler_params=pltpu.CompilerParams(
            dimension_semantics=("arbitrary",),
            vmem_limit_bytes=100 * 1024 * 1024,
        ),
    )(x, W, b2)
    return out.reshape(B, S, N)
